# Initial kernel scaffold; baseline (speedup 1.0000x reference)
import jax
import jax.numpy as jnp
from jax.experimental import pallas as pl


def kernel(x, edge_index, W1, b1, W2, b2, W3, b3, Wo1, bo1, Wo2, bo2):
    raise NotImplementedError("write your pallas kernel here")


# speedup vs baseline: 7.5808x; 7.5808x over previous
import functools

import jax
import jax.numpy as jnp
from jax import lax
from jax.experimental import pallas as pl
from jax.experimental.pallas import tpu as pltpu
from jax.experimental.pallas import tpu_sc as plsc

N = 10000
E = 320000
NC, NS = 2, 16
CHUNK = 128
EP = 327680
PAD = EP - E
ACC_ROWS = 10240
TRASH = N
NB = 2
NBD = 4
IW = 16
MAIN_CH = EP // NS // CHUNK
DEG_CH = EP // (NC * NS) // CHUNK
RB = 1000
NRB = N // RB
STRIPE = 624

_mesh = plsc.VectorSubcoreMesh(
    core_axis_name="c", subcore_axis_name="s", num_cores=NC, num_subcores=NS
)


@functools.partial(
    pl.kernel,
    out_type=jax.ShapeDtypeStruct((NC, N, 128), jnp.float32),
    mesh=_mesh,
    scratch_types=[
        pltpu.VMEM((DEG_CH, CHUNK), jnp.int32),
        pltpu.VMEM((CHUNK, 128), jnp.float32),
        pltpu.VMEM_SHARED((ACC_ROWS, 128), jnp.float32),
        pltpu.SemaphoreType.DMA,
    ],
)
def _deg_kernel(dst_hbm, ones_hbm, zeros_hbm, degp_hbm, dstv, onesv, acc, sem):
    c = lax.axis_index("c")
    s = lax.axis_index("s")
    pltpu.sync_copy(dst_hbm.at[c, s], dstv)
    pltpu.sync_copy(ones_hbm, onesv)
    pltpu.sync_copy(zeros_hbm, acc.at[pl.ds(s * (ACC_ROWS // NS), ACC_ROWS // NS)])
    plsc.subcore_barrier()

    def body(i, carry):
        hs = []
        for b in range(NBD):
            j = i * NBD + b
            hs.append(pltpu.async_copy(onesv, acc.at[dstv.at[j]], sem, add=True))
        for h in hs:
            h.wait()
        return carry

    lax.fori_loop(0, DEG_CH // NBD, body, 0)
    plsc.subcore_barrier()
    pltpu.sync_copy(
        acc.at[pl.ds(s * STRIPE, STRIPE)],
        degp_hbm.at[c, pl.ds(s * STRIPE, STRIPE)],
    )

    @pl.when(s == NS - 1)
    def _():
        pltpu.sync_copy(
            acc.at[pl.ds(NS * STRIPE, N - NS * STRIPE)],
            degp_hbm.at[c, pl.ds(NS * STRIPE, N - NS * STRIPE)],
        )


@functools.partial(
    pl.kernel,
    out_type=jax.ShapeDtypeStruct((NC, N, 128), jnp.float32),
    mesh=_mesh,
    scratch_types=[
        pltpu.VMEM((IW, CHUNK), jnp.int32),
        pltpu.VMEM((IW, CHUNK), jnp.int32),
        pltpu.VMEM((NB, CHUNK, 128), jnp.float32),
        pltpu.VMEM_SHARED((ACC_ROWS, 128), jnp.float32),
        pltpu.SemaphoreType.DMA,
        pltpu.SemaphoreType.DMA,
    ],
)
def _scatter_kernel(src_hbm, dst_hbm, hs_hbm, zeros_hbm, agg_hbm,
                    srcv, dstv, rows, acc, gsem, ssem):
    c = lax.axis_index("c")
    s = lax.axis_index("s")
    pltpu.sync_copy(zeros_hbm, acc.at[pl.ds(s * (ACC_ROWS // NS), ACC_ROWS // NS)])
    plsc.subcore_barrier()

    def window(w, carry):
        pltpu.sync_copy(src_hbm.at[c, s, pl.ds(w * IW, IW)], srcv)
        pltpu.sync_copy(dst_hbm.at[s, pl.ds(w * IW, IW)], dstv)

        def body(i, carry2):
            g = []
            for b in range(NB):
                j = i * NB + b
                g.append(pltpu.async_copy(hs_hbm.at[srcv.at[j]], rows.at[b], gsem))
            sc = []
            for b in range(NB):
                j = i * NB + b
                g[b].wait()
                sc.append(
                    pltpu.async_copy(rows.at[b], acc.at[dstv.at[j]], ssem, add=True))
            for h in sc:
                h.wait()
            return carry2

        return lax.fori_loop(0, IW // NB, body, carry)

    lax.fori_loop(0, MAIN_CH // IW, window, 0)
    plsc.subcore_barrier()
    pltpu.sync_copy(
        acc.at[pl.ds(s * STRIPE, STRIPE)],
        agg_hbm.at[c, pl.ds(s * STRIPE, STRIPE)],
    )

    @pl.when(s == NS - 1)
    def _():
        pltpu.sync_copy(
            acc.at[pl.ds(NS * STRIPE, N - NS * STRIPE)],
            agg_hbm.at[c, pl.ds(NS * STRIPE, N - NS * STRIPE)],
        )


def _tc1_body(x_ref, w_ref, dp0_ref, dp1_ref, hs_ref, dinv_ref):
    deg = dp0_ref[:, 0:1] + dp1_ref[:, 0:1] + 1.0
    dinv = lax.rsqrt(deg)
    h = jnp.dot(x_ref[...], w_ref[0], preferred_element_type=jnp.float32)
    hs_ref[...] = dinv * h
    dinv_ref[...] = jnp.broadcast_to(dinv, (RB, 128))


def _tc1(x, w1h, degp_flat):
    return pl.pallas_call(
        _tc1_body,
        grid=(NC, NRB),
        in_specs=[
            pl.BlockSpec((RB, 128), lambda c, i: (i, 0)),
            pl.BlockSpec((1, 128, 128), lambda c, i: (c, 0, 0)),
            pl.BlockSpec((RB, 128), lambda c, i: (i, 0)),
            pl.BlockSpec((RB, 128), lambda c, i: (NRB + i, 0)),
        ],
        out_specs=[
            pl.BlockSpec((RB, 128), lambda c, i: (c * NRB + i, 0)),
            pl.BlockSpec((RB, 128), lambda c, i: (i, 0)),
        ],
        out_shape=[
            jax.ShapeDtypeStruct((NC * N, 128), jnp.float32),
            jax.ShapeDtypeStruct((N, 128), jnp.float32),
        ],
    )(x, w1h, degp_flat, degp_flat)


def _tcmid_body(aggL_ref, aggR_ref, hsL_ref, hsR_ref, dinv_ref, b_ref,
                w_ref, out_ref):
    d = dinv_ref[...]
    actL = jnp.maximum(d * (aggL_ref[...] + hsL_ref[...]) + b_ref[0], 0.0)
    actR = jnp.maximum(d * (aggR_ref[...] + hsR_ref[...]) + b_ref[1], 0.0)
    h = (jnp.dot(actL, w_ref[0, 0], preferred_element_type=jnp.float32)
         + jnp.dot(actR, w_ref[0, 1], preferred_element_type=jnp.float32))
    out_ref[...] = d * h


def _tcmid(agg_flat, hs_flat, dinv_bc, b2, w4):
    return pl.pallas_call(
        _tcmid_body,
        grid=(NC, NRB),
        in_specs=[
            pl.BlockSpec((RB, 128), lambda c, i: (i, 0)),
            pl.BlockSpec((RB, 128), lambda c, i: (NRB + i, 0)),
            pl.BlockSpec((RB, 128), lambda c, i: (i, 0)),
            pl.BlockSpec((RB, 128), lambda c, i: (NRB + i, 0)),
            pl.BlockSpec((RB, 128), lambda c, i: (i, 0)),
            pl.BlockSpec((2, 128), lambda c, i: (0, 0)),
            pl.BlockSpec((1, 2, 128, 128), lambda c, i: (c, 0, 0, 0)),
        ],
        out_specs=pl.BlockSpec((RB, 128), lambda c, i: (c * NRB + i, 0)),
        out_shape=jax.ShapeDtypeStruct((NC * N, 128), jnp.float32),
    )(agg_flat, agg_flat, hs_flat, hs_flat, dinv_bc, b2, w4)


def _tcfin_body(aggL_ref, aggR_ref, hsL_ref, hsR_ref, dinv_ref, b_ref,
                wo1_ref, bo1_ref, wo2_ref, bo2_ref, out_ref):
    d = dinv_ref[...]
    actL = jnp.maximum(d * (aggL_ref[...] + hsL_ref[...]) + b_ref[0], 0.0)
    actR = jnp.maximum(d * (aggR_ref[...] + hsR_ref[...]) + b_ref[1], 0.0)
    t = (jnp.dot(actL, wo1_ref[0:128], preferred_element_type=jnp.float32)
         + jnp.dot(actR, wo1_ref[128:256], preferred_element_type=jnp.float32)
         + bo1_ref[0])
    out_ref[...] = jnp.dot(t, wo2_ref[...], preferred_element_type=jnp.float32) + bo2_ref[0]


def _tcfin(agg_flat, hs_flat, dinv_bc, b2, wo1, bo1, wo2, bo2):
    return pl.pallas_call(
        _tcfin_body,
        grid=(NRB,),
        in_specs=[
            pl.BlockSpec((RB, 128), lambda i: (i, 0)),
            pl.BlockSpec((RB, 128), lambda i: (NRB + i, 0)),
            pl.BlockSpec((RB, 128), lambda i: (i, 0)),
            pl.BlockSpec((RB, 128), lambda i: (NRB + i, 0)),
            pl.BlockSpec((RB, 128), lambda i: (i, 0)),
            pl.BlockSpec((2, 128), lambda i: (0, 0)),
            pl.BlockSpec((256, 256), lambda i: (0, 0)),
            pl.BlockSpec((1, 256), lambda i: (0, 0)),
            pl.BlockSpec((256, 128), lambda i: (0, 0)),
            pl.BlockSpec((1, 128), lambda i: (0, 0)),
        ],
        out_specs=pl.BlockSpec((RB, 128), lambda i: (i, 0)),
        out_shape=jax.ShapeDtypeStruct((N, 128), jnp.float32),
    )(agg_flat, agg_flat, hs_flat, hs_flat, dinv_bc, b2, wo1, bo1, wo2, bo2)


def kernel(x, edge_index, W1, b1, W2, b2, W3, b3, Wo1, bo1, Wo2, bo2):
    src = edge_index[0].astype(jnp.int32)
    dst = edge_index[1].astype(jnp.int32)
    src_p = jnp.concatenate([src, jnp.zeros((PAD,), jnp.int32)])
    dst_p = jnp.concatenate([dst, jnp.full((PAD,), TRASH, jnp.int32)])
    src2 = jnp.stack([src_p, src_p + N]).reshape(NC, NS, MAIN_CH, CHUNK)
    dst_main = dst_p.reshape(NS, MAIN_CH, CHUNK)
    dst_deg = dst_p.reshape(NC, NS, DEG_CH, CHUNK)
    ones128 = jnp.ones((CHUNK, 128), jnp.float32)
    zeros128 = jnp.zeros((ACC_ROWS // NS, 128), jnp.float32)

    degp = _deg_kernel(dst_deg, ones128, zeros128)
    degp_flat = degp.reshape(NC * N, 128)

    w1h = W1.reshape(128, 2, 128).transpose(1, 0, 2)
    w2_4 = W2.reshape(2, 128, 2, 128).transpose(2, 0, 1, 3)
    w3_4 = W3.reshape(2, 128, 2, 128).transpose(2, 0, 1, 3)

    hs1, dinv_bc = _tc1(x, w1h, degp_flat)
    agg1 = _scatter_kernel(src2, dst_main, hs1, zeros128).reshape(NC * N, 128)
    hs2 = _tcmid(agg1, hs1, dinv_bc, b1.reshape(2, 128), w2_4)
    agg2 = _scatter_kernel(src2, dst_main, hs2, zeros128).reshape(NC * N, 128)
    hs3 = _tcmid(agg2, hs2, dinv_bc, b2.reshape(2, 128), w3_4)
    agg3 = _scatter_kernel(src2, dst_main, hs3, zeros128).reshape(NC * N, 128)
    return _tcfin(agg3, hs3, dinv_bc, b3.reshape(2, 128), Wo1,
                  bo1.reshape(1, 256), Wo2, bo2.reshape(1, 128))


# speedup vs baseline: 8.0327x; 1.0596x over previous
import functools

import jax
import jax.numpy as jnp
from jax import lax
from jax.experimental import pallas as pl
from jax.experimental.pallas import tpu as pltpu
from jax.experimental.pallas import tpu_sc as plsc

N = 10000
E = 320000
NC, NS = 2, 16
CHUNK = 64
DCHUNK = 128
EP = 327680
PAD = EP - E
ACC_ROWS = 10240
TRASH = N
NB = 4
NBD = 8
IW = 32
MAIN_CH = EP // NS // CHUNK
DEG_CH = EP // (NC * NS) // DCHUNK
RB = 1000
NRB = N // RB
STRIPE = 624

_mesh = plsc.VectorSubcoreMesh(
    core_axis_name="c", subcore_axis_name="s", num_cores=NC, num_subcores=NS
)


@functools.partial(
    pl.kernel,
    out_type=jax.ShapeDtypeStruct((NC, N, 128), jnp.float32),
    mesh=_mesh,
    scratch_types=[
        pltpu.VMEM((DEG_CH, DCHUNK), jnp.int32),
        pltpu.VMEM((DCHUNK, 128), jnp.float32),
        pltpu.VMEM_SHARED((ACC_ROWS, 128), jnp.float32),
        pltpu.SemaphoreType.DMA,
    ],
)
def _deg_kernel(dst_hbm, ones_hbm, zeros_hbm, degp_hbm, dstv, onesv, acc, sem):
    c = lax.axis_index("c")
    s = lax.axis_index("s")
    pltpu.sync_copy(dst_hbm.at[c, s], dstv)
    pltpu.sync_copy(ones_hbm, onesv)
    pltpu.sync_copy(zeros_hbm, acc.at[pl.ds(s * (ACC_ROWS // NS), ACC_ROWS // NS)])
    plsc.subcore_barrier()

    def body(i, carry):
        hs = []
        for b in range(NBD):
            j = i * NBD + b
            hs.append(pltpu.async_copy(onesv, acc.at[dstv.at[j]], sem, add=True))
        for h in hs:
            h.wait()
        return carry

    lax.fori_loop(0, DEG_CH // NBD, body, 0)
    plsc.subcore_barrier()
    pltpu.sync_copy(
        acc.at[pl.ds(s * STRIPE, STRIPE)],
        degp_hbm.at[c, pl.ds(s * STRIPE, STRIPE)],
    )

    @pl.when(s == NS - 1)
    def _():
        pltpu.sync_copy(
            acc.at[pl.ds(NS * STRIPE, N - NS * STRIPE)],
            degp_hbm.at[c, pl.ds(NS * STRIPE, N - NS * STRIPE)],
        )


@functools.partial(
    pl.kernel,
    out_type=jax.ShapeDtypeStruct((NC, N, 128), jnp.float32),
    mesh=_mesh,
    scratch_types=[
        pltpu.VMEM((IW, CHUNK), jnp.int32),
        pltpu.VMEM((IW, CHUNK), jnp.int32),
        pltpu.VMEM((NB, CHUNK, 128), jnp.float32),
        pltpu.VMEM_SHARED((ACC_ROWS, 128), jnp.float32),
        pltpu.SemaphoreType.DMA,
    ] + [pltpu.SemaphoreType.DMA] * NB,
)
def _scatter_kernel(src_hbm, dst_hbm, hs_hbm, zeros_hbm, agg_hbm,
                    srcv, dstv, rows, acc, gsem, *ssems):
    c = lax.axis_index("c")
    s = lax.axis_index("s")
    pltpu.sync_copy(zeros_hbm, acc.at[pl.ds(s * (ACC_ROWS // NS), ACC_ROWS // NS)])
    plsc.subcore_barrier()
    nbat = IW // NB

    def window(w, carry):
        pltpu.sync_copy(src_hbm.at[c, s, pl.ds(w * IW, IW)], srcv)
        pltpu.sync_copy(dst_hbm.at[s, pl.ds(w * IW, IW)], dstv)

        def body(i, carry2):
            @pl.when(w * nbat + i > 0)
            def _():
                for b in range(NB):
                    pltpu.make_async_copy(
                        zeros_hbm.at[pl.ds(0, CHUNK)], rows.at[b], ssems[b]).wait()
            g = []
            for b in range(NB):
                j = i * NB + b
                g.append(pltpu.async_copy(hs_hbm.at[srcv.at[j]], rows.at[b], gsem))
            for b in range(NB):
                j = i * NB + b
                g[b].wait()
                pltpu.async_copy(rows.at[b], acc.at[dstv.at[j]], ssems[b], add=True)
            return carry2

        return lax.fori_loop(0, nbat, body, carry)

    lax.fori_loop(0, MAIN_CH // IW, window, 0)
    for b in range(NB):
        pltpu.make_async_copy(
            zeros_hbm.at[pl.ds(0, CHUNK)], rows.at[b], ssems[b]).wait()
    plsc.subcore_barrier()
    pltpu.sync_copy(
        acc.at[pl.ds(s * STRIPE, STRIPE)],
        agg_hbm.at[c, pl.ds(s * STRIPE, STRIPE)],
    )

    @pl.when(s == NS - 1)
    def _():
        pltpu.sync_copy(
            acc.at[pl.ds(NS * STRIPE, N - NS * STRIPE)],
            agg_hbm.at[c, pl.ds(NS * STRIPE, N - NS * STRIPE)],
        )


def _tc1_body(x_ref, w_ref, dp0_ref, dp1_ref, hs_ref, dinv_ref):
    deg = dp0_ref[:, 0:1] + dp1_ref[:, 0:1] + 1.0
    dinv = lax.rsqrt(deg)
    h = jnp.dot(x_ref[...], w_ref[0], preferred_element_type=jnp.float32)
    hs_ref[...] = dinv * h
    dinv_ref[...] = jnp.broadcast_to(dinv, (RB, 128))


def _tc1(x, w1h, degp_flat):
    return pl.pallas_call(
        _tc1_body,
        grid=(NC, NRB),
        in_specs=[
            pl.BlockSpec((RB, 128), lambda c, i: (i, 0)),
            pl.BlockSpec((1, 128, 128), lambda c, i: (c, 0, 0)),
            pl.BlockSpec((RB, 128), lambda c, i: (i, 0)),
            pl.BlockSpec((RB, 128), lambda c, i: (NRB + i, 0)),
        ],
        out_specs=[
            pl.BlockSpec((RB, 128), lambda c, i: (c * NRB + i, 0)),
            pl.BlockSpec((RB, 128), lambda c, i: (i, 0)),
        ],
        out_shape=[
            jax.ShapeDtypeStruct((NC * N, 128), jnp.float32),
            jax.ShapeDtypeStruct((N, 128), jnp.float32),
        ],
    )(x, w1h, degp_flat, degp_flat)


def _tcmid_body(aggL_ref, aggR_ref, hsL_ref, hsR_ref, dinv_ref, b_ref,
                w_ref, out_ref):
    d = dinv_ref[...]
    actL = jnp.maximum(d * (aggL_ref[...] + hsL_ref[...]) + b_ref[0], 0.0)
    actR = jnp.maximum(d * (aggR_ref[...] + hsR_ref[...]) + b_ref[1], 0.0)
    h = (jnp.dot(actL, w_ref[0, 0], preferred_element_type=jnp.float32)
         + jnp.dot(actR, w_ref[0, 1], preferred_element_type=jnp.float32))
    out_ref[...] = d * h


def _tcmid(agg_flat, hs_flat, dinv_bc, b2, w4):
    return pl.pallas_call(
        _tcmid_body,
        grid=(NC, NRB),
        in_specs=[
            pl.BlockSpec((RB, 128), lambda c, i: (i, 0)),
            pl.BlockSpec((RB, 128), lambda c, i: (NRB + i, 0)),
            pl.BlockSpec((RB, 128), lambda c, i: (i, 0)),
            pl.BlockSpec((RB, 128), lambda c, i: (NRB + i, 0)),
            pl.BlockSpec((RB, 128), lambda c, i: (i, 0)),
            pl.BlockSpec((2, 128), lambda c, i: (0, 0)),
            pl.BlockSpec((1, 2, 128, 128), lambda c, i: (c, 0, 0, 0)),
        ],
        out_specs=pl.BlockSpec((RB, 128), lambda c, i: (c * NRB + i, 0)),
        out_shape=jax.ShapeDtypeStruct((NC * N, 128), jnp.float32),
    )(agg_flat, agg_flat, hs_flat, hs_flat, dinv_bc, b2, w4)


def _tcfin_body(aggL_ref, aggR_ref, hsL_ref, hsR_ref, dinv_ref, b_ref,
                wo1_ref, bo1_ref, wo2_ref, bo2_ref, out_ref):
    d = dinv_ref[...]
    actL = jnp.maximum(d * (aggL_ref[...] + hsL_ref[...]) + b_ref[0], 0.0)
    actR = jnp.maximum(d * (aggR_ref[...] + hsR_ref[...]) + b_ref[1], 0.0)
    t = (jnp.dot(actL, wo1_ref[0:128], preferred_element_type=jnp.float32)
         + jnp.dot(actR, wo1_ref[128:256], preferred_element_type=jnp.float32)
         + bo1_ref[0])
    out_ref[...] = jnp.dot(t, wo2_ref[...], preferred_element_type=jnp.float32) + bo2_ref[0]


def _tcfin(agg_flat, hs_flat, dinv_bc, b2, wo1, bo1, wo2, bo2):
    return pl.pallas_call(
        _tcfin_body,
        grid=(NRB,),
        in_specs=[
            pl.BlockSpec((RB, 128), lambda i: (i, 0)),
            pl.BlockSpec((RB, 128), lambda i: (NRB + i, 0)),
            pl.BlockSpec((RB, 128), lambda i: (i, 0)),
            pl.BlockSpec((RB, 128), lambda i: (NRB + i, 0)),
            pl.BlockSpec((RB, 128), lambda i: (i, 0)),
            pl.BlockSpec((2, 128), lambda i: (0, 0)),
            pl.BlockSpec((256, 256), lambda i: (0, 0)),
            pl.BlockSpec((1, 256), lambda i: (0, 0)),
            pl.BlockSpec((256, 128), lambda i: (0, 0)),
            pl.BlockSpec((1, 128), lambda i: (0, 0)),
        ],
        out_specs=pl.BlockSpec((RB, 128), lambda i: (i, 0)),
        out_shape=jax.ShapeDtypeStruct((N, 128), jnp.float32),
    )(agg_flat, agg_flat, hs_flat, hs_flat, dinv_bc, b2, wo1, bo1, wo2, bo2)


def kernel(x, edge_index, W1, b1, W2, b2, W3, b3, Wo1, bo1, Wo2, bo2):
    src = edge_index[0].astype(jnp.int32)
    dst = edge_index[1].astype(jnp.int32)
    src_p = jnp.concatenate([src, jnp.zeros((PAD,), jnp.int32)])
    dst_p = jnp.concatenate([dst, jnp.full((PAD,), TRASH, jnp.int32)])
    src2 = jnp.stack([src_p, src_p + N]).reshape(NC, NS, MAIN_CH, CHUNK)
    dst_main = dst_p.reshape(NS, MAIN_CH, CHUNK)
    dst_deg = dst_p.reshape(NC, NS, DEG_CH, DCHUNK)
    ones128 = jnp.ones((DCHUNK, 128), jnp.float32)
    zeros128 = jnp.zeros((ACC_ROWS // NS, 128), jnp.float32)

    degp = _deg_kernel(dst_deg, ones128, zeros128)
    degp_flat = degp.reshape(NC * N, 128)

    w1h = W1.reshape(128, 2, 128).transpose(1, 0, 2)
    w2_4 = W2.reshape(2, 128, 2, 128).transpose(2, 0, 1, 3)
    w3_4 = W3.reshape(2, 128, 2, 128).transpose(2, 0, 1, 3)

    hs1, dinv_bc = _tc1(x, w1h, degp_flat)
    agg1 = _scatter_kernel(src2, dst_main, hs1, zeros128).reshape(NC * N, 128)
    hs2 = _tcmid(agg1, hs1, dinv_bc, b1.reshape(2, 128), w2_4)
    agg2 = _scatter_kernel(src2, dst_main, hs2, zeros128).reshape(NC * N, 128)
    hs3 = _tcmid(agg2, hs2, dinv_bc, b2.reshape(2, 128), w3_4)
    agg3 = _scatter_kernel(src2, dst_main, hs3, zeros128).reshape(NC * N, 128)
    return _tcfin(agg3, hs3, dinv_bc, b3.reshape(2, 128), Wo1,
                  bo1.reshape(1, 256), Wo2, bo2.reshape(1, 128))
